# baseline (speedup 1.0000x reference)
import functools

import jax
import jax.numpy as jnp
from jax import lax
from jax.experimental import pallas as pl
from jax.experimental.pallas import tpu as pltpu
from jax.experimental.pallas import tpu_sc as plsc

NUM_CORES = 2
NUM_SUBCORES = 16
NW = NUM_CORES * NUM_SUBCORES
CHUNK = 8
NBUF = 15


@functools.lru_cache(maxsize=None)
def _make_sc_gather(b: int, s: int, d: int):
    n_rows = b * s
    n_per_w = n_rows // NW
    n_chunks = n_per_w // CHUNK
    mesh = plsc.VectorSubcoreMesh(core_axis_name="c", subcore_axis_name="s")

    @functools.partial(
        pl.kernel,
        mesh=mesh,
        out_type=jax.ShapeDtypeStruct((n_rows, d), jnp.float32),
        scratch_types=[
            pltpu.VMEM((n_per_w,), jnp.int32),
            pltpu.VMEM((NBUF, CHUNK, d), jnp.float32),
            pltpu.SemaphoreType.DMA((NBUF,)),
            pltpu.SemaphoreType.DMA((NBUF,)),
        ],
    )
    def k(idx_hbm, table_hbm, out_hbm, idx_v, rows_v, gsem, wsem):
        wid = lax.axis_index("s") * NUM_CORES + lax.axis_index("c")
        base = wid * n_per_w
        pltpu.sync_copy(idx_hbm.at[base // s, pl.ds(base % s, n_per_w)], idx_v)

        def gather(c, buf):
            return pltpu.async_copy(
                table_hbm.at[idx_v.at[pl.ds(c * CHUNK, CHUNK)]],
                rows_v.at[buf], gsem.at[buf])

        def write(c, buf):
            return pltpu.async_copy(
                rows_v.at[buf],
                out_hbm.at[pl.ds(base + c * CHUNK, CHUNK)], wsem.at[buf])

        gd = [gather(buf, buf) for buf in range(NBUF)]
        wd = [None] * NBUF
        for c in range(n_chunks):
            buf = c % NBUF
            gd[buf].wait()
            wd[buf] = write(c, buf)
            pn = c - 1 + NBUF
            if c >= 1 and pn < n_chunks:
                pb = (c - 1) % NBUF
                wd[pb].wait()
                gd[pb] = gather(pn, pb)
        for buf in range(NBUF):
            if wd[buf] is not None:
                wd[buf].wait()

    return k


def kernel(input_ids, W):
    b, s = input_ids.shape
    out = _make_sc_gather(b, s, W.shape[1])(input_ids, W)
    return out.reshape(b, s, W.shape[1])


# speedup vs baseline: 1.0523x; 1.0523x over previous
import functools

import jax
import jax.numpy as jnp
from jax import lax
from jax.experimental import pallas as pl
from jax.experimental.pallas import tpu as pltpu
from jax.experimental.pallas import tpu_sc as plsc

NUM_CORES = 2
NUM_SUBCORES = 16
NW = NUM_CORES * NUM_SUBCORES
CHUNK = 16
NBUF = 7


@functools.lru_cache(maxsize=None)
def _make_sc_gather(b: int, s: int, d: int):
    n_rows = b * s
    n_per_w = n_rows // NW
    n_chunks = n_per_w // CHUNK
    mesh = plsc.VectorSubcoreMesh(core_axis_name="c", subcore_axis_name="s")

    @functools.partial(
        pl.kernel,
        mesh=mesh,
        out_type=jax.ShapeDtypeStruct((n_rows, d), jnp.float32),
        scratch_types=[
            pltpu.VMEM((n_per_w,), jnp.int32),
            pltpu.VMEM((NBUF, CHUNK, d), jnp.float32),
            pltpu.SemaphoreType.DMA((NBUF,)),
            pltpu.SemaphoreType.DMA((NBUF,)),
        ],
    )
    def k(idx_hbm, table_hbm, out_hbm, idx_v, rows_v, gsem, wsem):
        wid = lax.axis_index("s") * NUM_CORES + lax.axis_index("c")
        base = wid * n_per_w
        pltpu.sync_copy(idx_hbm.at[base // s, pl.ds(base % s, n_per_w)], idx_v)

        def gather(c, buf):
            return pltpu.make_async_copy(
                table_hbm.at[idx_v.at[pl.ds(c * CHUNK, CHUNK)]],
                rows_v.at[buf], gsem.at[buf])

        def write(c, buf):
            return pltpu.make_async_copy(
                rows_v.at[buf],
                out_hbm.at[pl.ds(base + c * CHUNK, CHUNK)], wsem.at[buf])

        for buf in range(NBUF):
            gather(buf, buf).start()

        def body(c, _):
            buf = lax.rem(c, NBUF)
            gather(c, buf).wait()
            write(c, buf).start()
            pc = c - 1

            @pl.when((pc >= 0) & (pc + NBUF < n_chunks))
            def _():
                pbuf = lax.rem(pc, NBUF)
                write(pc, pbuf).wait()
                gather(pc + NBUF, pbuf).start()

            return 0

        lax.fori_loop(0, n_chunks, body, 0)
        for t in range(NBUF):
            c = n_chunks - NBUF + t
            write(c, c % NBUF).wait()

    return k


def kernel(input_ids, W):
    b, s = input_ids.shape
    out = _make_sc_gather(b, s, W.shape[1])(input_ids, W)
    return out.reshape(b, s, W.shape[1])
